# baseline (speedup 1.0000x reference)
import functools

import jax
import jax.numpy as jnp
from jax import lax
from jax.experimental import pallas as pl
from jax.experimental.pallas import tpu as pltpu
from jax.experimental.pallas import tpu_sc as plsc

B = 4
S = 2048
D = 1024
NC = 2
NS = 16
NW = NC * NS
SEQ_PER_W = S // NW
CH = 16
NSUB = SEQ_PER_W // CH
NSTEP = NSUB * B
NBUF = 5
DIST = 3

_mesh = plsc.VectorSubcoreMesh(core_axis_name="c", subcore_axis_name="s")


@functools.partial(
    pl.kernel,
    out_type=jax.ShapeDtypeStruct((B * S, D), jnp.float32),
    mesh=_mesh,
    scratch_types=(
        [pltpu.VMEM((NSTEP, CH), jnp.int32)]
        + [pltpu.VMEM((CH, D), jnp.float32) for _ in range(NBUF)]
        + [pltpu.VMEM((CH, D), jnp.float32) for _ in range(2)]
        + [pltpu.SemaphoreType.DMA for _ in range(NBUF)]
        + [pltpu.SemaphoreType.DMA for _ in range(NBUF)]
        + [pltpu.SemaphoreType.DMA for _ in range(2)]
        + [pltpu.SemaphoreType.DMA]
    ),
)
def _sc_embed(ids_hbm, tok_hbm, pos_hbm, out_hbm, idx_v, *bufs):
    tok_v = bufs[0:NBUF]
    pos_v = bufs[NBUF:NBUF + 2]
    gsem = bufs[NBUF + 2:NBUF + 2 + NBUF]
    wsem = bufs[NBUF + 2 + NBUF:NBUF + 2 + 2 * NBUF]
    psem = bufs[NBUF + 2 + 2 * NBUF:NBUF + 4 + 2 * NBUF]
    isem = bufs[NBUF + 4 + 2 * NBUF]

    wid = lax.axis_index("s") * NC + lax.axis_index("c")
    seq_base = wid * SEQ_PER_W

    pos_d = [None, None]
    pos_d[0] = pltpu.async_copy(pos_hbm.at[pl.ds(seq_base, CH)], pos_v[0], psem[0])
    id_d = []
    for g in range(NSTEP):
        sub, b = divmod(g, B)
        id_d.append(
            pltpu.async_copy(
                ids_hbm.at[b, pl.ds(seq_base + sub * CH, CH)], idx_v.at[g], isem
            )
        )
    for d in id_d:
        d.wait()

    def gather(g):
        return pltpu.async_copy(
            tok_hbm.at[idx_v.at[g]], tok_v[g % NBUF], gsem[g % NBUF]
        )

    gat_d = [None] * NSTEP
    wr_d = [None] * NSTEP
    for g in range(DIST):
        gat_d[g] = gather(g)

    for g in range(NSTEP):
        nb = g % NBUF
        sub, b = divmod(g, B)
        if b == 0:
            pos_d[sub % 2].wait()
            if sub + 1 < NSUB:
                nxt = (sub + 1) % 2
                pos_d[nxt] = pltpu.async_copy(
                    pos_hbm.at[pl.ds(seq_base + (sub + 1) * CH, CH)],
                    pos_v[nxt], psem[nxt],
                )
        gat_d[g].wait()

        tv, pv = tok_v[nb], pos_v[sub % 2]

        def _add(r, _, tv=tv, pv=pv):
            for j in range(D // 16):
                col = j * 16
                plsc.addupdate(tv.at[r, pl.ds(col, 16)], pv[r, pl.ds(col, 16)])
            return _

        lax.fori_loop(0, CH, _add, None, unroll=False)

        wr_d[g] = pltpu.async_copy(
            tv, out_hbm.at[pl.ds(b * S + seq_base + sub * CH, CH)], wsem[nb]
        )
        if g + DIST < NSTEP:
            if g - (NBUF - DIST) >= 0:
                wr_d[g - (NBUF - DIST)].wait()
            gat_d[g + DIST] = gather(g + DIST)

    for t in range(NSTEP - NBUF, NSTEP):
        wr_d[t].wait()


def kernel(token_ids, tok_table, pos_table):
    out = _sc_embed(token_ids.astype(jnp.int32), tok_table, pos_table)
    return out.reshape(B, S, D)


# speedup vs baseline: 1.7197x; 1.7197x over previous
import functools

import jax
import jax.numpy as jnp
from jax import lax
from jax.experimental import pallas as pl
from jax.experimental.pallas import tpu as pltpu
from jax.experimental.pallas import tpu_sc as plsc

B = 4
S = 2048
D = 1024
NC = 2
NS = 16
NW = NC * NS
SEQ_PER_W = S // NW
CH = 16
NSUB = SEQ_PER_W // CH
NSTEP = NSUB * B
NBUF = 5
DIST = 3

_mesh = plsc.VectorSubcoreMesh(core_axis_name="c", subcore_axis_name="s")


@functools.partial(
    pl.kernel,
    out_type=jax.ShapeDtypeStruct((B * S, D), jnp.float32),
    mesh=_mesh,
    scratch_types=(
        [pltpu.VMEM((NSTEP, CH), jnp.int32)]
        + [pltpu.VMEM((CH, D), jnp.float32) for _ in range(NBUF)]
        + [pltpu.VMEM((CH, D), jnp.float32) for _ in range(2)]
        + [pltpu.SemaphoreType.DMA for _ in range(NBUF)]
        + [pltpu.SemaphoreType.DMA for _ in range(NBUF)]
        + [pltpu.SemaphoreType.DMA for _ in range(2)]
        + [pltpu.SemaphoreType.DMA]
    ),
)
def _sc_embed(ids_hbm, tok_hbm, pos_hbm, out_hbm, idx_v, *bufs):
    tok_v = bufs[0:NBUF]
    pos_v = bufs[NBUF:NBUF + 2]
    gsem = bufs[NBUF + 2:NBUF + 2 + NBUF]
    wsem = bufs[NBUF + 2 + NBUF:NBUF + 2 + 2 * NBUF]
    psem = bufs[NBUF + 2 + 2 * NBUF:NBUF + 4 + 2 * NBUF]
    isem = bufs[NBUF + 4 + 2 * NBUF]

    wid = lax.axis_index("s") * NC + lax.axis_index("c")
    seq_base = wid * SEQ_PER_W

    pos_d = [None, None]
    pos_d[0] = pltpu.async_copy(pos_hbm.at[pl.ds(seq_base, CH)], pos_v[0], psem[0])
    id_d = []
    for g in range(NSTEP):
        sub, b = divmod(g, B)
        id_d.append(
            pltpu.async_copy(
                ids_hbm.at[b, pl.ds(seq_base + sub * CH, CH)], idx_v.at[g], isem
            )
        )
    for d in id_d:
        d.wait()

    def gather(g):
        return pltpu.async_copy(
            tok_hbm.at[idx_v.at[g]], tok_v[g % NBUF], gsem[g % NBUF]
        )

    gat_d = [None] * NSTEP
    wr_d = [None] * NSTEP
    for g in range(DIST):
        gat_d[g] = gather(g)

    for g in range(NSTEP):
        nb = g % NBUF
        sub, b = divmod(g, B)
        if b == 0:
            pos_d[sub % 2].wait()
            if sub + 1 < NSUB:
                nxt = (sub + 1) % 2
                pos_d[nxt] = pltpu.async_copy(
                    pos_hbm.at[pl.ds(seq_base + (sub + 1) * CH, CH)],
                    pos_v[nxt], psem[nxt],
                )
        gat_d[g].wait()

        tv, pv = tok_v[nb], pos_v[sub % 2]

        @plsc.parallel_loop(0, D, step=16)
        def _add(col, tv=tv, pv=pv):
            for r in range(CH):
                plsc.addupdate(tv.at[r, pl.ds(col, 16)], pv[r, pl.ds(col, 16)])

        wr_d[g] = pltpu.async_copy(
            tv, out_hbm.at[pl.ds(b * S + seq_base + sub * CH, CH)], wsem[nb]
        )
        if g + DIST < NSTEP:
            if g - (NBUF - DIST) >= 0:
                wr_d[g - (NBUF - DIST)].wait()
            gat_d[g + DIST] = gather(g + DIST)

    for t in range(NSTEP - NBUF, NSTEP):
        wr_d[t].wait()


def kernel(token_ids, tok_table, pos_table):
    out = _sc_embed(token_ids.astype(jnp.int32), tok_table, pos_table)
    return out.reshape(B, S, D)
